# baseline (speedup 1.0000x reference)
import functools

import jax
import jax.numpy as jnp
from jax import lax
from jax.experimental import pallas as pl
from jax.experimental.pallas import tpu as pltpu
from jax.experimental.pallas import tpu_sc as plsc

_B = 4096
_H = 200
_D = 64
_LANES = 128
_NW = 32
_SLAB = 8 * 8 * _LANES
_WSTRIDE = 8 * _LANES
_DTSTRIDE = _NW * _WSTRIDE


def _emb_body(nc, idx3_hbm, tp_hbm, outf_hbm,
              idx_v, pidx_v, pb0, pb1, sl0, sl1, g0, g1, o0, o1):
    w = lax.axis_index("s") * nc + lax.axis_index("c")
    pltpu.sync_copy(idx3_hbm.at[:, w], idx_v)

    pbs = (pb0, pb1)
    sls = (sl0, sl1)
    gsems = (g0, g1)
    osems = (o0, o1)
    iota = lax.iota(jnp.int32, 16)
    rowvs = [iota + 16 * g for g in range(8)]

    def prep(h, b):
        for g in range(8):
            v = idx_v[h, pl.ds(16 * g, 16)]
            pidx_v[b, pl.ds(16 * g, 16)] = lax.shift_right_logical(v, 1)

    def gather_descr(b):
        return pltpu.make_async_copy(
            tp_hbm.at[pidx_v.at[b]], pbs[b], gsems[b])

    def out_descr(h, b):
        return pltpu.make_async_copy(
            sls[b], outf_hbm.at[h, :, w], osems[b])

    def transpose_select(h, b):
        pb, sl = pbs[b], sls[b]
        for g in range(8):
            v = idx_v[h, pl.ds(16 * g, 16)]
            colbase = lax.mul(lax.bitwise_and(v, 1), _D)
            rowv = rowvs[g]
            for dt in range(8):
                for di in range(8):
                    vals = plsc.load_gather(
                        pb, [rowv, colbase + (dt * 8 + di)])
                    sl[dt, di, pl.ds(16 * g, 16)] = vals

    prep(0, 0)
    gather_descr(0).start()

    def step(t):
        for b in range(2):
            h = 2 * t + b
            nb = 1 - b
            @pl.when(h + 1 < _H)
            def _fire_next():
                prep(h + 1, nb)
                gather_descr(nb).start()

            gather_descr(b).wait()
            @pl.when(h >= 2)
            def _wait_slab():
                out_descr(h - 2, b).wait()

            transpose_select(h, b)
            out_descr(h, b).start()

    pl.loop(0, _H // 2)(step)
    out_descr(_H - 2, 0).wait()
    out_descr(_H - 1, 1).wait()


def kernel(inputs, table):
    batch, hist = inputs.shape
    vocab, dim = table.shape
    assert (batch, hist, dim) == (_B, _H, _D)

    info = plsc.get_sparse_core_info()
    nc, ns = info.num_cores, info.num_subcores
    assert nc * ns == _NW

    idx3 = inputs.T.reshape(_H, _NW, _LANES)
    tpairs = table.reshape(vocab // 2, 2 * _D)

    emb = functools.partial(
        pl.kernel,
        mesh=plsc.VectorSubcoreMesh(core_axis_name="c", subcore_axis_name="s"),
        out_type=jax.ShapeDtypeStruct((_H, 8, _NW, 8, _LANES), jnp.float32),
        scratch_types=[
            pltpu.VMEM((_H, _LANES), jnp.int32),
            pltpu.VMEM((2, _LANES), jnp.int32),
            pltpu.VMEM((_LANES, 2 * _D), jnp.float32),
            pltpu.VMEM((_LANES, 2 * _D), jnp.float32),
            pltpu.VMEM((8, 8, _LANES), jnp.float32),
            pltpu.VMEM((8, 8, _LANES), jnp.float32),
            pltpu.SemaphoreType.DMA,
            pltpu.SemaphoreType.DMA,
            pltpu.SemaphoreType.DMA,
            pltpu.SemaphoreType.DMA,
        ],
        compiler_params=pltpu.CompilerParams(
            use_tc_tiling_on_sc=False, needs_layout_passes=False),
    )(functools.partial(_emb_body, nc))

    out5 = emb(idx3, tpairs)
    return out5.transpose(2, 4, 0, 1, 3).reshape(_B, _H, _D)


# speedup vs baseline: 1.5811x; 1.5811x over previous
import functools

import jax
import jax.numpy as jnp
from jax import lax
from jax.experimental import pallas as pl
from jax.experimental.pallas import tpu as pltpu
from jax.experimental.pallas import tpu_sc as plsc

_DIM = 64
_GATHER = 128
_CHUNK = 512
_GPC = _CHUNK // _GATHER


def _emb_body(nc, nchunk, ngather, per_w,
              idx_hbm, table_hbm, out_hbm, idx_v, buf0, buf1, g0, g1, o0, o1):
    wid = lax.axis_index("s") * nc + lax.axis_index("c")
    base = wid * per_w
    pltpu.sync_copy(idx_hbm.at[pl.ds(wid * ngather, ngather)], idx_v)

    bufs = (buf0, buf1)
    gsems = (g0, g1)
    osems = (o0, o1)

    def gather_descr(c, j, b):
        return pltpu.make_async_copy(
            table_hbm.at[idx_v.at[c * _GPC + j]],
            bufs[b].at[pl.ds(j * _GATHER, _GATHER)],
            gsems[b])

    def out_descr(c, b):
        return pltpu.make_async_copy(
            bufs[b], out_hbm.at[pl.ds(base + c * _CHUNK, _CHUNK)], osems[b])

    for j in range(_GPC):
        gather_descr(0, j, 0).start()

    def step(g):
        for b in range(2):
            c = 2 * g + b
            nb = 1 - b
            if b == 0:
                @pl.when(g >= 1)
                def _wait_prev():
                    out_descr(c - 1, nb).wait()

                for j in range(_GPC):
                    gather_descr(c + 1, j, nb).start()
            else:
                @pl.when(g < (nchunk - 2) // 2)
                def _fire_next():
                    out_descr(c - 1, nb).wait()
                    for j in range(_GPC):
                        gather_descr(c + 1, j, nb).start()
            for j in range(_GPC):
                gather_descr(c, j, b).wait()
            out_descr(c, b).start()

    pl.loop(0, nchunk // 2)(step)
    out_descr(nchunk - 2, 0).wait()
    out_descr(nchunk - 1, 1).wait()


def kernel(inputs, table):
    batch, hist = inputs.shape
    vocab, dim = table.shape
    assert dim == _DIM
    total = batch * hist

    info = plsc.get_sparse_core_info()
    nc, ns = info.num_cores, info.num_subcores
    nw = nc * ns
    per_w = total // nw
    assert per_w % _CHUNK == 0
    nchunk = per_w // _CHUNK
    ngather = per_w // _GATHER

    idx2d = inputs.reshape(total // _GATHER, _GATHER)

    emb = functools.partial(
        pl.kernel,
        mesh=plsc.VectorSubcoreMesh(core_axis_name="c", subcore_axis_name="s"),
        out_type=jax.ShapeDtypeStruct((total, _DIM), jnp.float32),
        scratch_types=[
            pltpu.VMEM((ngather, _GATHER), jnp.int32),
            pltpu.VMEM((_CHUNK, _DIM), jnp.float32),
            pltpu.VMEM((_CHUNK, _DIM), jnp.float32),
            pltpu.SemaphoreType.DMA,
            pltpu.SemaphoreType.DMA,
            pltpu.SemaphoreType.DMA,
            pltpu.SemaphoreType.DMA,
        ],
        compiler_params=pltpu.CompilerParams(use_tc_tiling_on_sc=False),
    )(functools.partial(_emb_body, nc, nchunk, ngather, per_w))

    out = emb(idx2d, table)
    return out.reshape(batch, hist, _DIM)
